# baseline (speedup 1.0000x reference)
import jax
import jax.numpy as jnp
from jax.experimental import pallas as pl
from jax.experimental.pallas import tpu as pltpu


_TM = 512


def _interp_tile(s_ref, x_ref, o_ref):
    i = pl.program_id(0)
    o_ref[pl.ds(i * _TM, _TM), :] = jnp.dot(
        s_ref[0], x_ref[...], preferred_element_type=jnp.float32)


def kernel(x_coarse, interp_matrix):
    m, k = interp_matrix.shape
    n = x_coarse.shape[1]
    steps = m // _TM
    return pl.pallas_call(
        _interp_tile,
        grid=(steps,),
        in_specs=[
            pl.BlockSpec((1, _TM, k), lambda i: (i, 0, 0)),
            pl.BlockSpec(memory_space=pltpu.MemorySpace.VMEM),
        ],
        out_specs=pl.BlockSpec(memory_space=pltpu.MemorySpace.VMEM),
        out_shape=jax.ShapeDtypeStruct((m, n), jnp.float32),
    )(interp_matrix.reshape(steps, _TM, k), x_coarse)


# speedup vs baseline: 1.0026x; 1.0026x over previous
import jax
import jax.numpy as jnp
from jax.experimental import pallas as pl
from jax.experimental.pallas import tpu as pltpu


_TM = 512


def _interp_tile(s_ref, x_ref, o_ref):
    o_ref[...] = jnp.dot(s_ref[0], x_ref[...],
                         preferred_element_type=jnp.float32)


def kernel(x_coarse, interp_matrix):
    m, k = interp_matrix.shape
    n = x_coarse.shape[1]
    steps = m // _TM
    return pl.pallas_call(
        _interp_tile,
        grid=(steps,),
        in_specs=[
            pl.BlockSpec((1, _TM, k), lambda i: (i, 0, 0)),
            pl.BlockSpec(memory_space=pltpu.MemorySpace.VMEM),
        ],
        out_specs=pl.BlockSpec((_TM, n), lambda i: (i, 0)),
        out_shape=jax.ShapeDtypeStruct((m, n), jnp.float32),
        compiler_params=pltpu.CompilerParams(
            dimension_semantics=("arbitrary",)),
    )(interp_matrix.reshape(steps, _TM, k), x_coarse)
